# baseline (device time: 199210 ns/iter reference)
import jax
import jax.numpy as jnp
from jax import lax
from jax.experimental import pallas as pl
from jax.experimental.pallas import tpu as pltpu

N_DEV = 16
SQ = 1024
D = 1024
HQ_PER = 8
DH = 128
HD = HQ_PER * DH
CHUNK = SQ // N_DEV
SCALE = 0.08838834764831843
WINDOW = 128


def _ring_allreduce(partial):

    def body(p_ref, out_ref, acc_ref, comm_ref, rs_send, rs_recv, ag_send, ag_recv):
        my = lax.axis_index("i")
        left = lax.rem(my - 1 + N_DEV, N_DEV)
        right = lax.rem(my + 1, N_DEV)

        barrier = pltpu.get_barrier_semaphore()
        for nbr in (left, right):
            pl.semaphore_signal(
                barrier, inc=1, device_id=(nbr,),
                device_id_type=pl.DeviceIdType.MESH,
            )
        pl.semaphore_wait(barrier, 2)

        acc_ref[...] = p_ref[pl.ds(my * CHUNK, CHUNK), :]
        for s in range(N_DEV - 1):
            rdma = pltpu.make_async_remote_copy(
                src_ref=acc_ref,
                dst_ref=comm_ref.at[s],
                send_sem=rs_send.at[s],
                recv_sem=rs_recv.at[s],
                device_id=(right,),
                device_id_type=pl.DeviceIdType.MESH,
            )
            rdma.start()
            rdma.wait()
            c = lax.rem(my - s - 1 + N_DEV, N_DEV)
            acc_ref[...] = comm_ref[s] + p_ref[pl.ds(c * CHUNK, CHUNK), :]
        own = lax.rem(my + 1, N_DEV)
        out_ref[pl.ds(own * CHUNK, CHUNK), :] = acc_ref[...]

        for s in range(N_DEV - 1):
            o = lax.rem(my + 1 - s + N_DEV, N_DEV)
            rdma = pltpu.make_async_remote_copy(
                src_ref=out_ref.at[pl.ds(o * CHUNK, CHUNK), :],
                dst_ref=out_ref.at[pl.ds(o * CHUNK, CHUNK), :],
                send_sem=ag_send.at[s],
                recv_sem=ag_recv.at[s],
                device_id=(right,),
                device_id_type=pl.DeviceIdType.MESH,
            )
            rdma.start()
            rdma.wait()

    return pl.pallas_call(
        body,
        out_shape=jax.ShapeDtypeStruct((SQ, D), jnp.float32),
        in_specs=[pl.BlockSpec(memory_space=pltpu.VMEM)],
        out_specs=pl.BlockSpec(memory_space=pltpu.VMEM),
        scratch_shapes=[
            pltpu.VMEM((CHUNK, D), jnp.float32),
            pltpu.VMEM((N_DEV - 1, CHUNK, D), jnp.float32),
            pltpu.SemaphoreType.DMA((N_DEV - 1,)),
            pltpu.SemaphoreType.DMA((N_DEV - 1,)),
            pltpu.SemaphoreType.DMA((N_DEV - 1,)),
            pltpu.SemaphoreType.DMA((N_DEV - 1,)),
        ],
        compiler_params=pltpu.CompilerParams(collective_id=0),
    )(partial)


def kernel(x, Wq, K_ext, V_ext, Wo):
    pos = lax.axis_index("i")

    Wq_my = lax.dynamic_slice(Wq, (0, pos * HD), (D, HD))
    Wo_my = lax.dynamic_slice(Wo, (pos * HD, 0), (HD, D))

    Q = (x[0] @ Wq_my).reshape(SQ, HQ_PER, DH)
    K = K_ext[0]
    V = V_ext[0]

    scores = jnp.einsum(
        "qhd,khd->hqk", Q, K, preferred_element_type=jnp.float32
    ) * SCALE
    qi = lax.broadcasted_iota(jnp.int32, (SQ, SQ), 0)
    ki = lax.broadcasted_iota(jnp.int32, (SQ, SQ), 1)
    mask = jnp.abs(qi - ki) <= WINDOW
    scores = jnp.where(mask[None], scores, -1e9)
    m = scores.max(axis=-1, keepdims=True)
    w = jnp.exp(scores - m)
    w = w / w.sum(axis=-1, keepdims=True)
    ctx = jnp.einsum(
        "hqk,khd->qhd", w, V, preferred_element_type=jnp.float32
    ).reshape(SQ, HD)
    partial = ctx @ Wo_my

    out = _ring_allreduce(partial)
    return out[None]


# device time: 120317 ns/iter; 1.6557x vs baseline; 1.6557x over previous
import jax
import jax.numpy as jnp
from jax import lax
from jax.experimental import pallas as pl
from jax.experimental.pallas import tpu as pltpu

N_DEV = 16
SQ = 1024
D = 1024
HQ_PER = 8
DH = 128
HD = HQ_PER * DH
SCALE = 0.08838834764831843
WINDOW = 128

RS_STEPS = [(1, 512), (4, 256), (2, 128), (8, 64)]
AG_STEPS = [(8, 64), (2, 128), (4, 256), (1, 512)]
WIRE_DTYPE = jnp.bfloat16


def _butterfly_allreduce(partial):

    def body(p_ref, out_ref, sbuf, r0, r1, r2, r3, a0, a1, a2, a3,
             rs_send, rs_recv, ag_send, ag_recv):
        my = lax.axis_index("i")

        barrier = pltpu.get_barrier_semaphore()
        for m in (1, 2, 4, 8):
            pl.semaphore_signal(
                barrier, inc=1,
                device_id=(jnp.bitwise_xor(my, m),),
                device_id_type=pl.DeviceIdType.MESH,
            )
        pl.semaphore_wait(barrier, 4)

        rs_rbufs = [r0, r1, r2, r3]
        ag_rbufs = [a0, a1, a2, a3]

        lo = jnp.int32(0)
        for j, (m, half) in enumerate(RS_STEPS):
            bit = jnp.bitwise_and(my // m, 1)
            keep_lo = lo + bit * half
            send_lo = lo + (1 - bit) * half
            src = p_ref if j == 0 else out_ref
            sbuf[pl.ds(0, half), :] = src[pl.ds(send_lo, half), :].astype(
                WIRE_DTYPE
            )
            rdma = pltpu.make_async_remote_copy(
                src_ref=sbuf.at[pl.ds(0, half), :],
                dst_ref=rs_rbufs[j],
                send_sem=rs_send.at[j],
                recv_sem=rs_recv.at[j],
                device_id=(jnp.bitwise_xor(my, m),),
                device_id_type=pl.DeviceIdType.MESH,
            )
            rdma.start()
            rdma.wait()
            out_ref[pl.ds(keep_lo, half), :] = (
                src[pl.ds(keep_lo, half), :]
                + rs_rbufs[j][...].astype(jnp.float32)
            )
            lo = keep_lo

        for j, (m, n) in enumerate(AG_STEPS):
            bit = jnp.bitwise_and(my // m, 1)
            merged_lo = lo - bit * n
            other_lo = merged_lo + (1 - bit) * n
            sbuf[pl.ds(0, n), :] = out_ref[pl.ds(lo, n), :].astype(WIRE_DTYPE)
            rdma = pltpu.make_async_remote_copy(
                src_ref=sbuf.at[pl.ds(0, n), :],
                dst_ref=ag_rbufs[j],
                send_sem=ag_send.at[j],
                recv_sem=ag_recv.at[j],
                device_id=(jnp.bitwise_xor(my, m),),
                device_id_type=pl.DeviceIdType.MESH,
            )
            rdma.start()
            rdma.wait()
            out_ref[pl.ds(other_lo, n), :] = ag_rbufs[j][...].astype(
                jnp.float32
            )
            lo = merged_lo

    return pl.pallas_call(
        body,
        out_shape=jax.ShapeDtypeStruct((SQ, D), jnp.float32),
        in_specs=[pl.BlockSpec(memory_space=pltpu.VMEM)],
        out_specs=pl.BlockSpec(memory_space=pltpu.VMEM),
        scratch_shapes=[
            pltpu.VMEM((512, D), WIRE_DTYPE),
            pltpu.VMEM((512, D), WIRE_DTYPE),
            pltpu.VMEM((256, D), WIRE_DTYPE),
            pltpu.VMEM((128, D), WIRE_DTYPE),
            pltpu.VMEM((64, D), WIRE_DTYPE),
            pltpu.VMEM((64, D), WIRE_DTYPE),
            pltpu.VMEM((128, D), WIRE_DTYPE),
            pltpu.VMEM((256, D), WIRE_DTYPE),
            pltpu.VMEM((512, D), WIRE_DTYPE),
            pltpu.SemaphoreType.DMA((4,)),
            pltpu.SemaphoreType.DMA((4,)),
            pltpu.SemaphoreType.DMA((4,)),
            pltpu.SemaphoreType.DMA((4,)),
        ],
        compiler_params=pltpu.CompilerParams(collective_id=0),
    )(partial)


def kernel(x, Wq, K_ext, V_ext, Wo):
    pos = lax.axis_index("i")

    Wq_my = lax.dynamic_slice(Wq, (0, pos * HD), (D, HD))
    Wo_my = lax.dynamic_slice(Wo, (pos * HD, 0), (HD, D))

    Q = (x[0] @ Wq_my).reshape(SQ, HQ_PER, DH)
    K = K_ext[0]
    V = V_ext[0]

    scores = jnp.einsum(
        "qhd,khd->hqk", Q, K, preferred_element_type=jnp.float32
    ) * SCALE
    qi = lax.broadcasted_iota(jnp.int32, (SQ, SQ), 0)
    ki = lax.broadcasted_iota(jnp.int32, (SQ, SQ), 1)
    mask = jnp.abs(qi - ki) <= WINDOW
    scores = jnp.where(mask[None], scores, -1e9)
    m = scores.max(axis=-1, keepdims=True)
    w = jnp.exp(scores - m)
    w = w / w.sum(axis=-1, keepdims=True)
    ctx = jnp.einsum(
        "hqk,khd->qhd", w, V, preferred_element_type=jnp.float32
    ).reshape(SQ, HD)
    partial = ctx @ Wo_my

    out = _butterfly_allreduce(partial)
    return out[None]


# device time: 109049 ns/iter; 1.8268x vs baseline; 1.1033x over previous
import jax
import jax.numpy as jnp
from jax import lax
from jax.experimental import pallas as pl
from jax.experimental.pallas import tpu as pltpu

N_DEV = 16
SQ = 1024
D = 1024
HQ_PER = 8
DH = 128
HD = HQ_PER * DH
SCALE = 0.08838834764831843
WINDOW = 128

RS_STEPS = [(1, 512), (4, 256), (2, 128), (8, 64)]
AG_STEPS = [(8, 64), (2, 128), (4, 256), (1, 512)]
WIRE_DTYPE = jnp.bfloat16


def _butterfly_allreduce(partial):

    def body(p_ref, out_ref, sbuf, r0, r1, r2, r3, a0, a1, a2, a3,
             rs_send, rs_recv, ag_send, ag_recv):
        my = lax.axis_index("i")

        barrier = pltpu.get_barrier_semaphore()
        for m in (1, 2, 4, 8):
            pl.semaphore_signal(
                barrier, inc=1,
                device_id=(jnp.bitwise_xor(my, m),),
                device_id_type=pl.DeviceIdType.MESH,
            )
        pl.semaphore_wait(barrier, 4)

        rs_rbufs = [r0, r1, r2, r3]
        ag_rbufs = [a0, a1, a2, a3]

        lo = jnp.int32(0)
        for j, (m, half) in enumerate(RS_STEPS):
            bit = jnp.bitwise_and(my // m, 1)
            keep_lo = lo + bit * half
            send_lo = lo + (1 - bit) * half
            src = p_ref if j == 0 else out_ref
            sbuf[pl.ds(0, half), :] = src[pl.ds(send_lo, half), :].astype(
                WIRE_DTYPE
            )
            rdma = pltpu.make_async_remote_copy(
                src_ref=sbuf.at[pl.ds(0, half), :],
                dst_ref=rs_rbufs[j],
                send_sem=rs_send.at[j],
                recv_sem=rs_recv.at[j],
                device_id=(jnp.bitwise_xor(my, m),),
                device_id_type=pl.DeviceIdType.MESH,
            )
            rdma.start()
            rdma.wait()
            out_ref[pl.ds(keep_lo, half), :] = (
                src[pl.ds(keep_lo, half), :]
                + rs_rbufs[j][...].astype(jnp.float32)
            )
            lo = keep_lo

        for j, (m, n) in enumerate(AG_STEPS):
            bit = jnp.bitwise_and(my // m, 1)
            merged_lo = lo - bit * n
            other_lo = merged_lo + (1 - bit) * n
            sbuf[pl.ds(0, n), :] = out_ref[pl.ds(lo, n), :].astype(WIRE_DTYPE)
            rdma = pltpu.make_async_remote_copy(
                src_ref=sbuf.at[pl.ds(0, n), :],
                dst_ref=ag_rbufs[j],
                send_sem=ag_send.at[j],
                recv_sem=ag_recv.at[j],
                device_id=(jnp.bitwise_xor(my, m),),
                device_id_type=pl.DeviceIdType.MESH,
            )
            rdma.start()
            rdma.wait()
            out_ref[pl.ds(other_lo, n), :] = ag_rbufs[j][...].astype(
                jnp.float32
            )
            lo = merged_lo

    return pl.pallas_call(
        body,
        out_shape=jax.ShapeDtypeStruct((SQ, D), jnp.float32),
        in_specs=[pl.BlockSpec(memory_space=pltpu.VMEM)],
        out_specs=pl.BlockSpec(memory_space=pltpu.VMEM),
        scratch_shapes=[
            pltpu.VMEM((512, D), WIRE_DTYPE),
            pltpu.VMEM((512, D), WIRE_DTYPE),
            pltpu.VMEM((256, D), WIRE_DTYPE),
            pltpu.VMEM((128, D), WIRE_DTYPE),
            pltpu.VMEM((64, D), WIRE_DTYPE),
            pltpu.VMEM((64, D), WIRE_DTYPE),
            pltpu.VMEM((128, D), WIRE_DTYPE),
            pltpu.VMEM((256, D), WIRE_DTYPE),
            pltpu.VMEM((512, D), WIRE_DTYPE),
            pltpu.SemaphoreType.DMA((4,)),
            pltpu.SemaphoreType.DMA((4,)),
            pltpu.SemaphoreType.DMA((4,)),
            pltpu.SemaphoreType.DMA((4,)),
        ],
        compiler_params=pltpu.CompilerParams(collective_id=0),
    )(partial)


def kernel(x, Wq, K_ext, V_ext, Wo):
    pos = lax.axis_index("i")

    Wq_my = lax.dynamic_slice(Wq, (0, pos * HD), (D, HD))
    Wo_my = lax.dynamic_slice(Wo, (pos * HD, 0), (HD, D))

    xb = x[0].astype(jnp.bfloat16)
    Q = (
        jnp.dot(xb, Wq_my.astype(jnp.bfloat16), preferred_element_type=jnp.float32)
        .astype(jnp.bfloat16)
        .reshape(SQ, HQ_PER, DH)
    )
    K = K_ext[0].astype(jnp.bfloat16)
    V = V_ext[0].astype(jnp.bfloat16)

    scores = jnp.einsum(
        "qhd,khd->hqk", Q, K, preferred_element_type=jnp.float32
    ) * SCALE
    qi = lax.broadcasted_iota(jnp.int32, (SQ, SQ), 0)
    ki = lax.broadcasted_iota(jnp.int32, (SQ, SQ), 1)
    mask = jnp.abs(qi - ki) <= WINDOW
    scores = jnp.where(mask[None], scores, -1e9)
    m = scores.max(axis=-1, keepdims=True)
    w = jnp.exp(scores - m)
    w = (w / w.sum(axis=-1, keepdims=True)).astype(jnp.bfloat16)
    ctx = (
        jnp.einsum("hqk,khd->qhd", w, V, preferred_element_type=jnp.float32)
        .astype(jnp.bfloat16)
        .reshape(SQ, HD)
    )
    partial = jnp.dot(
        ctx, Wo_my.astype(jnp.bfloat16), preferred_element_type=jnp.float32
    )

    out = _butterfly_allreduce(partial)
    return out[None]
